# Initial kernel scaffold; baseline (speedup 1.0000x reference)
import functools

import jax
import jax.numpy as jnp
from jax import lax
from jax.experimental import pallas as pl
from jax.experimental.pallas import tpu as pltpu
from jax.experimental.pallas import tpu_sc as plsc

BATCH = 4096
MAX_LANDMARKS = 50
EMBED_DIM = 128

NC = 2
NS = 16
L = 16
NW = NC * NS

B_TOTAL = BATCH * MAX_LANDMARKS
B_PER_W = B_TOTAL // NW
CHUNK = 256
N_CHUNKS = B_PER_W // CHUNK

_MESH = plsc.VectorSubcoreMesh(core_axis_name="c", subcore_axis_name="s")


def _rsqrt16(s):
    i = plsc.bitcast(s, jnp.int32)
    i = jnp.int32(0x5F3759DF) - (i >> 1)
    y = plsc.bitcast(i, jnp.float32)
    half_s = s * 0.5
    for _ in range(3):
        y = y * (1.5 - half_s * y * y)
    return y


def _body(idx_hbm, table_hbm, out_hbm, idx_v, rows_v, sem):
    wid = lax.axis_index("s") * NC + lax.axis_index("c")
    base = wid * B_PER_W

    def chunk_step(g, _):
        row0 = base + g * CHUNK
        pltpu.sync_copy(idx_hbm.at[pl.ds(row0, CHUNK)], idx_v)
        pltpu.async_copy(table_hbm.at[idx_v], rows_v, sem).wait()

        def row_step(r, _):
            chunks = [rows_v[r, pl.ds(16 * k, 16)] for k in range(8)]
            acc = chunks[0] * chunks[0]
            for k in range(1, 8):
                acc = acc + chunks[k] * chunks[k]
            total = jnp.broadcast_to(jnp.sum(acc, axis=0), (16,))
            inv = _rsqrt16(total)
            for k in range(8):
                rows_v[r, pl.ds(16 * k, 16)] = chunks[k] * inv
            return 0

        lax.fori_loop(0, CHUNK, row_step, 0, unroll=2)
        pltpu.sync_copy(rows_v, out_hbm.at[pl.ds(row0, CHUNK)])
        return 0

    lax.fori_loop(0, N_CHUNKS, chunk_step, 0)


@jax.jit
def _gather_normalize(indices_flat, table):
    return pl.kernel(
        _body,
        out_type=jax.ShapeDtypeStruct((B_TOTAL, EMBED_DIM), jnp.float32),
        mesh=_MESH,
        scratch_types=[
            pltpu.VMEM((CHUNK,), jnp.int32),
            pltpu.VMEM((CHUNK, EMBED_DIM), jnp.float32),
            pltpu.SemaphoreType.DMA,
        ],
    )(indices_flat, table)


def kernel(indices, table):
    features = _gather_normalize(indices.reshape(-1), table)
    features = features.reshape(BATCH, MAX_LANDMARKS, EMBED_DIM)
    mask = jnp.zeros(indices.shape, dtype=bool)
    return features, mask


# speedup vs baseline: 2.1364x; 2.1364x over previous
import jax
import jax.numpy as jnp
from jax import lax
from jax.experimental import pallas as pl
from jax.experimental.pallas import tpu as pltpu
from jax.experimental.pallas import tpu_sc as plsc

BATCH = 4096
MAX_LANDMARKS = 50
EMBED_DIM = 128

NC = 2
NS = 16
NW = NC * NS

B_TOTAL = BATCH * MAX_LANDMARKS
CHUNK = 128
CPW = B_TOTAL // (NW * CHUNK)
B_PER_W = CPW * CHUNK


def _gather_body(idx_hbm, table_hbm, out_hbm, idx_v, rows_v, sem):
    wid = lax.axis_index("s") * NC + lax.axis_index("c")
    pltpu.sync_copy(idx_hbm.at[wid], idx_v)
    base = wid * B_PER_W

    def step(g, _):
        pltpu.async_copy(table_hbm.at[idx_v.at[g]], rows_v, sem).wait()
        pltpu.sync_copy(rows_v, out_hbm.at[pl.ds(base + g * CHUNK, CHUNK)])
        return 0

    lax.fori_loop(0, CPW, step, 0)


def _sc_gather(idx2d, table):
    return pl.kernel(
        _gather_body,
        out_type=jax.ShapeDtypeStruct((B_TOTAL, EMBED_DIM), jnp.float32),
        mesh=plsc.VectorSubcoreMesh(core_axis_name="c", subcore_axis_name="s"),
        scratch_types=[
            pltpu.VMEM((CPW, CHUNK), jnp.int32),
            pltpu.VMEM((CHUNK, EMBED_DIM), jnp.float32),
            pltpu.SemaphoreType.DMA,
        ],
    )(idx2d, table)


ROWS_BLK = 2048


def _norm_body(x_ref, o_ref):
    x = x_ref[...]
    s = jnp.sum(x * x, axis=1, keepdims=True)
    o_ref[...] = x * lax.rsqrt(s)


def _tc_normalize(raw):
    return pl.pallas_call(
        _norm_body,
        grid=(B_TOTAL // ROWS_BLK,),
        in_specs=[pl.BlockSpec((ROWS_BLK, EMBED_DIM), lambda i: (i, 0))],
        out_specs=pl.BlockSpec((ROWS_BLK, EMBED_DIM), lambda i: (i, 0)),
        out_shape=jax.ShapeDtypeStruct((B_TOTAL, EMBED_DIM), jnp.float32),
    )(raw)


@jax.jit
def _run(indices, table):
    idx2d = indices.reshape(NW, CPW, CHUNK)
    raw = _sc_gather(idx2d, table)
    features = _tc_normalize(raw).reshape(BATCH, MAX_LANDMARKS, EMBED_DIM)
    mask = jnp.zeros(indices.shape, dtype=bool)
    return features, mask


def kernel(indices, table):
    return _run(indices, table)


# speedup vs baseline: 2.3241x; 1.0878x over previous
import jax
import jax.numpy as jnp
from jax import lax
from jax.experimental import pallas as pl
from jax.experimental.pallas import tpu as pltpu
from jax.experimental.pallas import tpu_sc as plsc

BATCH = 4096
MAX_LANDMARKS = 50
EMBED_DIM = 128

NC = 2
NS = 16
NW = NC * NS

B_TOTAL = BATCH * MAX_LANDMARKS
CHUNK = 128
CPW = B_TOTAL // (NW * CHUNK)
B_PER_W = CPW * CHUNK


NBUF = 5
ROUNDS = CPW // NBUF


def _gather_body(idx_hbm, table_hbm, out_hbm, idx_v, *scr):
    rows = scr[:NBUF]
    gsem = scr[NBUF:2 * NBUF]
    osem = scr[2 * NBUF:3 * NBUF]
    wid = lax.axis_index("s") * NC + lax.axis_index("c")
    pltpu.sync_copy(idx_hbm.at[wid], idx_v)
    base = wid * B_PER_W

    def out_slice(c):
        return out_hbm.at[pl.ds(base + c * CHUNK, CHUNK)]

    for b in range(NBUF):
        pltpu.async_copy(table_hbm.at[idx_v.at[b]], rows[b], gsem[b])

    def round_(k, _):
        c0 = k * NBUF
        for b in range(NBUF):
            c = c0 + b
            pltpu.make_async_copy(table_hbm.at[idx_v.at[c]], rows[b], gsem[b]).wait()
            pltpu.async_copy(rows[b], out_slice(c), osem[b])
        for b in range(NBUF):
            c = c0 + b
            pltpu.make_async_copy(rows[b], out_slice(c), osem[b]).wait()
            pltpu.async_copy(table_hbm.at[idx_v.at[c + NBUF]], rows[b], gsem[b])
        return 0

    lax.fori_loop(0, ROUNDS - 1, round_, 0)

    c0 = (ROUNDS - 1) * NBUF
    for b in range(NBUF):
        c = c0 + b
        pltpu.make_async_copy(table_hbm.at[idx_v.at[c]], rows[b], gsem[b]).wait()
        pltpu.async_copy(rows[b], out_slice(c), osem[b])
    for b in range(NBUF):
        c = c0 + b
        pltpu.make_async_copy(rows[b], out_slice(c), osem[b]).wait()


def _sc_gather(idx2d, table):
    scratch = (
        [pltpu.VMEM((CPW, CHUNK), jnp.int32)]
        + [pltpu.VMEM((CHUNK, EMBED_DIM), jnp.float32) for _ in range(NBUF)]
        + [pltpu.SemaphoreType.DMA for _ in range(2 * NBUF)]
    )
    return pl.kernel(
        _gather_body,
        out_type=jax.ShapeDtypeStruct((B_TOTAL, EMBED_DIM), jnp.float32),
        mesh=plsc.VectorSubcoreMesh(core_axis_name="c", subcore_axis_name="s"),
        scratch_types=scratch,
    )(idx2d, table)


ROWS_BLK = 2048


def _norm_body(x_ref, o_ref):
    x = x_ref[...]
    s = jnp.sum(x * x, axis=1, keepdims=True)
    o_ref[...] = x * lax.rsqrt(s)


def _tc_normalize(raw):
    return pl.pallas_call(
        _norm_body,
        grid=(B_TOTAL // ROWS_BLK,),
        in_specs=[pl.BlockSpec((ROWS_BLK, EMBED_DIM), lambda i: (i, 0))],
        out_specs=pl.BlockSpec((ROWS_BLK, EMBED_DIM), lambda i: (i, 0)),
        out_shape=jax.ShapeDtypeStruct((B_TOTAL, EMBED_DIM), jnp.float32),
    )(raw)


@jax.jit
def _run(indices, table):
    idx2d = indices.reshape(NW, CPW, CHUNK)
    raw = _sc_gather(idx2d, table)
    features = _tc_normalize(raw).reshape(BATCH, MAX_LANDMARKS, EMBED_DIM)
    mask = jnp.zeros(indices.shape, dtype=bool)
    return features, mask


def kernel(indices, table):
    return _run(indices, table)


# speedup vs baseline: 2.5961x; 1.1171x over previous
import jax
import jax.numpy as jnp
from jax import lax
from jax.experimental import pallas as pl
from jax.experimental.pallas import tpu as pltpu
from jax.experimental.pallas import tpu_sc as plsc

BATCH = 4096
MAX_LANDMARKS = 50
EMBED_DIM = 128

NC = 2
NS = 16
NW = NC * NS

BPW = BATCH // NW
NBUF = 4
ROUNDS = BPW // NBUF


def _gather_body(idx_hbm, table_hbm, out_hbm, idx_v, *scr):
    rows = scr[:NBUF]
    gsem = scr[NBUF:2 * NBUF]
    osem = scr[2 * NBUF:3 * NBUF]
    wid = lax.axis_index("s") * NC + lax.axis_index("c")
    base = wid * BPW
    pltpu.sync_copy(idx_hbm.at[pl.ds(base, BPW)], idx_v)

    def gfire(i, b):
        pltpu.async_copy(table_hbm.at[idx_v.at[i]], rows[b], gsem[b])

    def gwait(i, b):
        pltpu.make_async_copy(table_hbm.at[idx_v.at[i]], rows[b], gsem[b]).wait()

    def ofire(i, b):
        pltpu.async_copy(rows[b], out_hbm.at[base + i], osem[b])

    def owait(i, b):
        pltpu.make_async_copy(rows[b], out_hbm.at[base + i], osem[b]).wait()

    for b in range(NBUF):
        gfire(b, b)

    def round_(k, _):
        i0 = k * NBUF
        for b in range(NBUF):
            gwait(i0 + b, b)
            ofire(i0 + b, b)
        for b in range(NBUF):
            owait(i0 + b, b)
            gfire(i0 + NBUF + b, b)
        return 0

    lax.fori_loop(0, ROUNDS - 1, round_, 0)

    i0 = (ROUNDS - 1) * NBUF
    for b in range(NBUF):
        gwait(i0 + b, b)
        ofire(i0 + b, b)
    for b in range(NBUF):
        owait(i0 + b, b)


def _sc_gather(indices, table):
    scratch = (
        [pltpu.VMEM((BPW, MAX_LANDMARKS), jnp.int32)]
        + [pltpu.VMEM((MAX_LANDMARKS, EMBED_DIM), jnp.float32) for _ in range(NBUF)]
        + [pltpu.SemaphoreType.DMA for _ in range(2 * NBUF)]
    )
    return pl.kernel(
        _gather_body,
        out_type=jax.ShapeDtypeStruct((BATCH, MAX_LANDMARKS, EMBED_DIM), jnp.float32),
        mesh=plsc.VectorSubcoreMesh(core_axis_name="c", subcore_axis_name="s"),
        scratch_types=scratch,
    )(indices, table)


GB = 16


def _norm_body(x_ref, o_ref):
    x = x_ref[...]
    s = jnp.sum(x * x, axis=-1, keepdims=True)
    o_ref[...] = x * lax.rsqrt(s)


def _tc_normalize(raw):
    blk = (GB, MAX_LANDMARKS, EMBED_DIM)
    return pl.pallas_call(
        _norm_body,
        grid=(BATCH // GB,),
        in_specs=[pl.BlockSpec(blk, lambda i: (i, 0, 0))],
        out_specs=pl.BlockSpec(blk, lambda i: (i, 0, 0)),
        out_shape=jax.ShapeDtypeStruct((BATCH, MAX_LANDMARKS, EMBED_DIM), jnp.float32),
    )(raw)


@jax.jit
def _run(indices, table):
    features = _tc_normalize(_sc_gather(indices, table))
    mask = jnp.zeros(indices.shape, dtype=bool)
    return features, mask


def kernel(indices, table):
    return _run(indices, table)


# speedup vs baseline: 3.1225x; 1.2028x over previous
import jax
import jax.numpy as jnp
from jax import lax
from jax.experimental import pallas as pl
from jax.experimental.pallas import tpu as pltpu
from jax.experimental.pallas import tpu_sc as plsc

BATCH = 4096
MAX_LANDMARKS = 50
EMBED_DIM = 128

NC = 2
NS = 16
NW = NC * NS

BPW = BATCH // NW
NBUF = 4
ROUNDS = BPW // NBUF


def _gather_body(idx_hbm, table_hbm, out_hbm, idx_v, *scr):
    rows = scr[:NBUF]
    gsem = scr[NBUF:2 * NBUF]
    osem = scr[2 * NBUF:3 * NBUF]
    wid = lax.axis_index("s") * NC + lax.axis_index("c")
    base = wid * BPW
    pltpu.sync_copy(idx_hbm.at[pl.ds(base, BPW)], idx_v)

    def gfire(i, b):
        pltpu.async_copy(table_hbm.at[idx_v.at[i]], rows[b], gsem[b])

    def gwait(i, b):
        pltpu.make_async_copy(table_hbm.at[idx_v.at[i]], rows[b], gsem[b]).wait()

    def ofire(i, b):
        pltpu.async_copy(rows[b], out_hbm.at[base + i], osem[b])

    def owait(i, b):
        pltpu.make_async_copy(rows[b], out_hbm.at[base + i], osem[b]).wait()

    for b in range(NBUF):
        gfire(b, b)

    def round_(k, _):
        i0 = k * NBUF
        for b in range(NBUF):
            gwait(i0 + b, b)
            ofire(i0 + b, b)
        for b in range(NBUF):
            owait(i0 + b, b)
            gfire(i0 + NBUF + b, b)
        return 0

    lax.fori_loop(0, ROUNDS - 1, round_, 0)

    i0 = (ROUNDS - 1) * NBUF
    for b in range(NBUF):
        gwait(i0 + b, b)
        ofire(i0 + b, b)
    for b in range(NBUF):
        owait(i0 + b, b)


def _sc_gather(indices, table):
    scratch = (
        [pltpu.VMEM((BPW, MAX_LANDMARKS), jnp.int32)]
        + [pltpu.VMEM((MAX_LANDMARKS, EMBED_DIM), jnp.float32) for _ in range(NBUF)]
        + [pltpu.SemaphoreType.DMA for _ in range(2 * NBUF)]
    )
    return pl.kernel(
        _gather_body,
        out_type=jax.ShapeDtypeStruct((BATCH, MAX_LANDMARKS, EMBED_DIM), jnp.float32),
        mesh=plsc.VectorSubcoreMesh(core_axis_name="c", subcore_axis_name="s"),
        scratch_types=scratch,
    )(indices, table)


GB = 32


def _norm_body(x_ref, o_ref):
    x = x_ref[...]
    s = jnp.sum(x * x, axis=-1, keepdims=True)
    o_ref[...] = x * lax.rsqrt(s)


def _tc_normalize(raw):
    blk = (GB, MAX_LANDMARKS, EMBED_DIM)
    return pl.pallas_call(
        _norm_body,
        grid=(BATCH // GB,),
        in_specs=[pl.BlockSpec(blk, lambda i: (i, 0, 0))],
        out_specs=pl.BlockSpec(blk, lambda i: (i, 0, 0)),
        out_shape=jax.ShapeDtypeStruct((BATCH, MAX_LANDMARKS, EMBED_DIM), jnp.float32),
        compiler_params=pltpu.CompilerParams(
            dimension_semantics=("parallel",),
        ),
    )(raw)


@jax.jit
def _run(indices, table):
    features = _tc_normalize(_sc_gather(indices, table))
    mask = jnp.zeros(indices.shape, dtype=bool)
    return features, mask


def kernel(indices, table):
    return _run(indices, table)


# speedup vs baseline: 4.1847x; 1.3402x over previous
import jax
import jax.numpy as jnp
from jax import lax
from jax.experimental import pallas as pl
from jax.experimental.pallas import tpu as pltpu
from jax.experimental.pallas import tpu_sc as plsc

BATCH = 4096
MAX_LANDMARKS = 50
EMBED_DIM = 128

NC = 2
NS = 16
NW = NC * NS
L = 16

BPW = BATCH // NW
NBUF = 4
ROWS_PAD = 64


def _rsqrt16(s):
    f = s * 0.0 + 1.0
    for e in (64, 32, 16, 8, 4, 2):
        big = s >= 2.0 ** e
        s = jnp.where(big, s * 2.0 ** -e, s)
        f = jnp.where(big, f * 2.0 ** (-e // 2), f)
        small = s < 4.0 * 2.0 ** -e
        s = jnp.where(small, s * 2.0 ** e, s)
        f = jnp.where(small, f * 2.0 ** (e // 2), f)
    y = 7.0 / 6.0 - s * (1.0 / 6.0)
    for _ in range(4):
        y = y * (1.5 - 0.5 * s * y * y)
    return y * f


def _normalize_item(buf):
    iota = lax.iota(jnp.int32, L)

    def group(g, _):
        def quad1(q, tot):
            for r in range(4):
                lane = q * 4 + r
                row = g * L + lane
                acc = None
                for k in range(8):
                    c = buf[row, pl.ds(L * k, L)]
                    acc = c * c if acc is None else acc + c * c
                for sh in (8, 4, 2, 1):
                    acc = acc + acc.at[iota ^ sh].get(mode="promise_in_bounds")
                tot = jnp.where(iota == lane, acc, tot)
            return tot

        tot = lax.fori_loop(0, 4, quad1, jnp.zeros((L,), jnp.float32))
        inv = _rsqrt16(tot)

        def quad2(q, _):
            for r in range(4):
                lane = q * 4 + r
                row = g * L + lane
                b = inv.at[iota * 0 + lane].get(mode="promise_in_bounds")
                for k in range(8):
                    buf[row, pl.ds(L * k, L)] = buf[row, pl.ds(L * k, L)] * b
            return 0

        lax.fori_loop(0, 4, quad2, 0)
        return 0

    lax.fori_loop(0, 4, group, 0)


def _gather_body(idx_hbm, table_hbm, out_hbm, idx_v, *scr):
    rows = scr[:NBUF]
    gsem = scr[NBUF:2 * NBUF]
    osem = scr[2 * NBUF:3 * NBUF]
    wid = lax.axis_index("s") * NC + lax.axis_index("c")
    base = wid * BPW
    pltpu.sync_copy(idx_hbm.at[pl.ds(base, BPW)], idx_v)

    def gfire(i, b):
        pltpu.async_copy(
            table_hbm.at[idx_v.at[i]], rows[b].at[pl.ds(0, MAX_LANDMARKS)], gsem[b])

    def gwait(i, b):
        pltpu.make_async_copy(
            table_hbm.at[idx_v.at[i]], rows[b].at[pl.ds(0, MAX_LANDMARKS)],
            gsem[b]).wait()

    def ofire(i, b):
        pltpu.async_copy(
            rows[b].at[pl.ds(0, MAX_LANDMARKS)], out_hbm.at[base + i], osem[b])

    def owait(i, b):
        pltpu.make_async_copy(
            rows[b].at[pl.ds(0, MAX_LANDMARKS)], out_hbm.at[base + i],
            osem[b]).wait()

    def visit(i, b, b2, refill, drain):
        gwait(i, b)
        _normalize_item(rows[b])
        ofire(i, b)
        if drain:
            owait(i - 2, b2)
        if refill:
            gfire(i + 2, b2)

    gfire(0, 0)
    gfire(1, 1)
    visit(0, 0, 2, True, False)
    visit(1, 1, 3, True, False)

    def round_(m, _):
        v0 = 2 + 4 * m
        for j in range(4):
            visit(v0 + j, (2 + j) % NBUF, j % NBUF, True, True)
        return 0

    lax.fori_loop(0, (BPW - NBUF) // NBUF, round_, 0)

    visit(BPW - 2, 2, 0, False, True)
    visit(BPW - 1, 3, 1, False, True)
    owait(BPW - 2, 2)
    owait(BPW - 1, 3)


def _sc_gather_normalize(indices, table):
    scratch = (
        [pltpu.VMEM((BPW, MAX_LANDMARKS), jnp.int32)]
        + [pltpu.VMEM((ROWS_PAD, EMBED_DIM), jnp.float32) for _ in range(NBUF)]
        + [pltpu.SemaphoreType.DMA for _ in range(2 * NBUF)]
    )
    return pl.kernel(
        _gather_body,
        out_type=jax.ShapeDtypeStruct((BATCH, MAX_LANDMARKS, EMBED_DIM), jnp.float32),
        mesh=plsc.VectorSubcoreMesh(core_axis_name="c", subcore_axis_name="s"),
        scratch_types=scratch,
    )(indices, table)


@jax.jit
def _run(indices, table):
    features = _sc_gather_normalize(indices, table)
    mask = jnp.zeros(indices.shape, dtype=bool)
    return features, mask


def kernel(indices, table):
    return _run(indices, table)
